# Initial kernel scaffold; baseline (speedup 1.0000x reference)
import jax
import jax.numpy as jnp
from jax.experimental import pallas as pl


def kernel(batch_hynm, batch_hynm_weights, table):
    raise NotImplementedError("write your pallas kernel here")


# speedup vs baseline: 2.3078x; 2.3078x over previous
import functools

import jax
import jax.numpy as jnp
from jax import lax
from jax.experimental import pallas as pl
from jax.experimental.pallas import tpu as pltpu
from jax.experimental.pallas import tpu_sc as plsc

_B = 16384
_L = 50
_D = 64
_NW = 32
_PER_W = _B // _NW
_C = 8
_NG = _PER_W // _C
_RPG = _C * _L
_GCH = 80


def _weight_slot(l):
  if l < 32:
    return l // 16, l % 16
  if l < 34:
    return 2, l - 32
  return 3, l - 34


def _body(idx_hbm, w_hbm, table_hbm, out_hbm, idx_v, w_v, rows_v, out_v, sem):
  wid = lax.axis_index("s") * 2 + lax.axis_index("c")

  def group(g, carry):
    b0 = wid * _PER_W + g * _C
    r0 = b0 * _L
    pltpu.sync_copy(idx_hbm.at[pl.ds(r0, _RPG)], idx_v)
    pltpu.sync_copy(w_hbm.at[pl.ds(r0, _RPG)], w_v)
    copies = [
        pltpu.async_copy(
            table_hbm.at[idx_v.at[pl.ds(j * _GCH, _GCH)]],
            rows_v.at[pl.ds(j * _GCH, _GCH)],
            sem,
        )
        for j in range(_RPG // _GCH)
    ]
    for cp in copies:
      cp.wait()

    def bstep(b, c):
      base = b * _L
      wv = [
          w_v[pl.ds(base, 16)],
          w_v[pl.ds(base + 16, 16)],
          w_v[pl.ds(base + 32, 16)],
          w_v[pl.ds(base + 34, 16)],
      ]
      accs = [jnp.zeros((16,), jnp.float32) for _ in range(4)]
      for l in range(_L):
        vr, lane = _weight_slot(l)
        ws = wv[vr][lane]
        for k in range(4):
          accs[k] = accs[k] + ws * rows_v[base + l, pl.ds(k * 16, 16)]
      for k in range(4):
        out_v[pl.ds(b * _D + k * 16, 16)] = accs[k]
      return c

    lax.fori_loop(0, _C, bstep, 0)
    pltpu.sync_copy(out_v, out_hbm.at[pl.ds(b0 * _D, _C * _D)])
    return carry

  lax.fori_loop(0, _NG, group, 0)


def kernel(batch_hynm, batch_hynm_weights, table):
  idx = batch_hynm.reshape(-1).astype(jnp.int32)
  w = batch_hynm_weights.reshape(-1)
  run = pl.kernel(
      _body,
      out_type=jax.ShapeDtypeStruct((_B * _D,), jnp.float32),
      mesh=plsc.VectorSubcoreMesh(core_axis_name="c", subcore_axis_name="s"),
      scratch_types=[
          pltpu.VMEM((_RPG,), jnp.int32),
          pltpu.VMEM((_RPG,), jnp.float32),
          pltpu.VMEM((_RPG, _D), jnp.float32),
          pltpu.VMEM((_C * _D,), jnp.float32),
          pltpu.SemaphoreType.DMA,
      ],
      compiler_params=pltpu.CompilerParams(use_tc_tiling_on_sc=False),
  )
  out = run(idx, w, table)
  return out.reshape(_B, _D)
